# Initial kernel scaffold; baseline (speedup 1.0000x reference)
import jax
import jax.numpy as jnp
from jax.experimental import pallas as pl


def kernel(components, levels, time_elapsed, comp_table, level_table):
    raise NotImplementedError("write your pallas kernel here")


# speedup vs baseline: 2.7755x; 2.7755x over previous
import jax
import jax.numpy as jnp
from jax.experimental import pallas as pl


def _onehot_body(comp_ref, lev_ref, t_ref, out_ref, *, n_comp, n_lev):
    R, D = out_ref.shape
    i = jax.lax.broadcasted_iota(jnp.int32, (R, D), 1)
    comp = comp_ref[...]
    lev = lev_ref[...]
    t = t_ref[...]
    one = jnp.float32(1.0)
    zero = jnp.float32(0.0)
    c_val = jnp.where(i == comp + 1, one, zero)
    l_val = jnp.where(i == lev + (n_comp + 1), one, zero)
    onehot = jnp.where(i <= n_comp, c_val, l_val)
    out_ref[...] = jnp.where(i == 0, t, onehot)


def kernel(components, levels, time_elapsed, comp_table, level_table):
    n_comp = comp_table.shape[1]
    n_lev = level_table.shape[1]
    D = 1 + n_comp + n_lev
    B, L = components.shape
    N = B * L

    comp = components.reshape(N, 1).astype(jnp.int32)
    lev = levels.reshape(N, 1).astype(jnp.int32)
    t = time_elapsed.reshape(N, 1)

    R = 1024
    assert N % R == 0
    grid = N // R

    import functools
    body = functools.partial(_onehot_body, n_comp=n_comp, n_lev=n_lev)

    out = pl.pallas_call(
        body,
        grid=(grid,),
        in_specs=[
            pl.BlockSpec((R, 1), lambda g: (g, 0)),
            pl.BlockSpec((R, 1), lambda g: (g, 0)),
            pl.BlockSpec((R, 1), lambda g: (g, 0)),
        ],
        out_specs=pl.BlockSpec((R, D), lambda g: (g, 0)),
        out_shape=jax.ShapeDtypeStruct((N, D), jnp.float32),
    )(comp, lev, t)
    return out.reshape(B, L, D)
